# baseline (speedup 1.0000x reference)
import functools

import jax
import jax.numpy as jnp
from jax import lax
from jax.experimental import pallas as pl
from jax.experimental.pallas import tpu as pltpu
from jax.experimental.pallas import tpu_sc as plsc

VOCAB = 10000
DIM = 128
N_OUT = 128
SEQ_PAD = 56


@functools.cache
def _make_gather(n_batch, seq):
    NC, NS = 2, 16
    NW = NC * NS
    n_idx = n_batch * seq
    b_per_w = n_idx // NW
    chunk = 400
    nbuf = 2
    n_chunks = b_per_w // chunk
    bat_per_chunk = chunk // seq
    mesh = plsc.VectorSubcoreMesh(core_axis_name="c", subcore_axis_name="s")

    @functools.partial(
        pl.kernel,
        mesh=mesh,
        out_type=jax.ShapeDtypeStruct((n_batch, SEQ_PAD, DIM), jnp.float32),
        scratch_types=[
            pltpu.VMEM((b_per_w,), jnp.int32),
            *[pltpu.VMEM((chunk + 8, DIM), jnp.float32) for _ in range(nbuf)],
            *[pltpu.SemaphoreType.DMA for _ in range(2 * nbuf)],
        ],
    )
    def gather_k(emb_hbm, idx_hbm, h_hbm, idx_v, *bufs_and_sems):
        rows = bufs_and_sems[:nbuf]
        gsem = bufs_and_sems[nbuf:2 * nbuf]
        wsem = bufs_and_sems[2 * nbuf:]
        wid = lax.axis_index("s") * NC + lax.axis_index("c")
        base = pl.multiple_of(wid * b_per_w, 8)
        base_b = wid * (b_per_w // seq)
        pltpu.sync_copy(idx_hbm.at[pl.ds(base, b_per_w)], idx_v)

        def gather_chunk(c, b):
            off = pl.multiple_of(c * chunk, 8)
            return pltpu.make_async_copy(
                emb_hbm.at[idx_v.at[pl.ds(off, chunk)]],
                rows[b].at[pl.ds(0, chunk)],
                gsem[b],
            )

        def write_chunk(c, b):
            bo = base_b + c * bat_per_chunk
            return [
                pltpu.make_async_copy(
                    rows[b].at[pl.ds(k * seq, SEQ_PAD)],
                    h_hbm.at[bo + k],
                    wsem[b],
                )
                for k in range(bat_per_chunk)
            ]

        for b in range(nbuf):
            gather_chunk(b, b).start()
        for c in range(n_chunks):
            b = c % nbuf
            gather_chunk(c, b).wait()
            for cp in write_chunk(c, b):
                cp.start()
            if c + nbuf < n_chunks:
                for cp in write_chunk(c, b):
                    cp.wait()
                gather_chunk(c + nbuf, b).start()
        for c in range(max(0, n_chunks - nbuf), n_chunks):
            for cp in write_chunk(c, c % nbuf):
                cp.wait()

    return gather_k


_BAT_BLOCK = 32


def _linear_body(h_ref, w_ref, b_ref, out_ref):
    seq = out_ref.shape[1]
    h2 = h_ref[...].reshape(_BAT_BLOCK * SEQ_PAD, DIM)
    acc = lax.dot_general(
        h2, w_ref[...],
        dimension_numbers=(((1,), (1,)), ((), ())),
        preferred_element_type=jnp.float32,
    ) + b_ref[...]
    out_ref[...] = acc.reshape(_BAT_BLOCK, SEQ_PAD, N_OUT)[:, :seq, :]


def _apply_linear(h, W, b, n_batch, seq):
    return pl.pallas_call(
        _linear_body,
        grid=(n_batch // _BAT_BLOCK,),
        in_specs=[
            pl.BlockSpec((_BAT_BLOCK, SEQ_PAD, DIM), lambda i: (i, 0, 0)),
            pl.BlockSpec((N_OUT, DIM), lambda i: (0, 0)),
            pl.BlockSpec((1, N_OUT), lambda i: (0, 0)),
        ],
        out_specs=pl.BlockSpec((_BAT_BLOCK, seq, N_OUT), lambda i: (i, 0, 0)),
        out_shape=jax.ShapeDtypeStruct((n_batch, seq, N_OUT), jnp.float32),
    )(h, W, b.reshape(1, N_OUT))


def kernel(x, emb, W, b):
    n_batch, seq = x.shape
    idx = x.reshape(-1).astype(jnp.int32)
    h = _make_gather(n_batch, seq)(emb, idx)
    return _apply_linear(h, W, b, n_batch, seq)


# speedup vs baseline: 2.9047x; 2.9047x over previous
import functools

import jax
import jax.numpy as jnp
from jax import lax
from jax.experimental import pallas as pl
from jax.experimental.pallas import tpu as pltpu
from jax.experimental.pallas import tpu_sc as plsc

VOCAB = 10000
DIM = 128
N_OUT = 128

_ROW_BLOCK = 1000


def _table_body(emb_ref, w_ref, b_ref, out_ref):
    acc = lax.dot_general(
        emb_ref[...], w_ref[...],
        dimension_numbers=(((1,), (1,)), ((), ())),
        preferred_element_type=jnp.float32,
    )
    out_ref[...] = acc + b_ref[...]


def _build_table(emb, W, b):
    grid = VOCAB // _ROW_BLOCK
    return pl.pallas_call(
        _table_body,
        grid=(grid,),
        in_specs=[
            pl.BlockSpec((_ROW_BLOCK, DIM), lambda i: (i, 0)),
            pl.BlockSpec((N_OUT, DIM), lambda i: (0, 0)),
            pl.BlockSpec((1, N_OUT), lambda i: (0, 0)),
        ],
        out_specs=pl.BlockSpec((_ROW_BLOCK, N_OUT), lambda i: (i, 0)),
        out_shape=jax.ShapeDtypeStruct((VOCAB, N_OUT), jnp.float32),
    )(emb, W, b.reshape(1, N_OUT))


@functools.cache
def _make_gather(n_idx):
    NC, NS = 2, 16
    NW = NC * NS
    b_per_w = n_idx // NW
    chunk = 200
    nbuf = 4
    n_chunks = b_per_w // chunk
    mesh = plsc.VectorSubcoreMesh(core_axis_name="c", subcore_axis_name="s")

    @functools.partial(
        pl.kernel,
        mesh=mesh,
        out_type=jax.ShapeDtypeStruct((n_idx, N_OUT), jnp.float32),
        scratch_types=[
            pltpu.VMEM((b_per_w,), jnp.int32),
            *[pltpu.VMEM((chunk, N_OUT), jnp.float32) for _ in range(nbuf)],
            *[pltpu.SemaphoreType.DMA for _ in range(2 * nbuf)],
        ],
    )
    def gather_k(table_hbm, idx_hbm, out_hbm, idx_v, *bufs_and_sems):
        rows = bufs_and_sems[:nbuf]
        gsem = bufs_and_sems[nbuf:2 * nbuf]
        wsem = bufs_and_sems[2 * nbuf:]
        wid = lax.axis_index("s") * NC + lax.axis_index("c")
        base = pl.multiple_of(wid * b_per_w, 8)
        pltpu.sync_copy(idx_hbm.at[pl.ds(base, b_per_w)], idx_v)

        def gather_chunk(c, b):
            off = pl.multiple_of(c * chunk, 8)
            return pltpu.make_async_copy(
                table_hbm.at[idx_v.at[pl.ds(off, chunk)]], rows[b], gsem[b]
            )

        def write_chunk(c, b):
            off = pl.multiple_of(base + c * chunk, 8)
            return pltpu.make_async_copy(
                rows[b], out_hbm.at[pl.ds(off, chunk)], wsem[b]
            )

        for b in range(nbuf):
            gather_chunk(b, b).start()
        for c in range(n_chunks):
            b = c % nbuf
            gather_chunk(c, b).wait()
            write_chunk(c, b).start()
            if c + nbuf < n_chunks:
                write_chunk(c, b).wait()
                gather_chunk(c + nbuf, b).start()
        for c in range(max(0, n_chunks - nbuf), n_chunks):
            write_chunk(c, c % nbuf).wait()

    return gather_k


def kernel(x, emb, W, b):
    n_batch, seq = x.shape
    table = _build_table(emb, W, b)
    idx = x.T.reshape(-1).astype(jnp.int32)
    out = _make_gather(idx.shape[0])(table, idx)
    return out.reshape(seq, n_batch, N_OUT).transpose(1, 0, 2)


# speedup vs baseline: 4.0507x; 1.3945x over previous
import functools

import jax
import jax.numpy as jnp
from jax import lax
from jax.experimental import pallas as pl
from jax.experimental.pallas import tpu as pltpu
from jax.experimental.pallas import tpu_sc as plsc

VOCAB = 10000
DIM = 128
N_OUT = 128

_ROW_BLOCK = 2000


def _table_body(emb_ref, w_ref, b_ref, out_ref):
    acc = lax.dot_general(
        emb_ref[...], w_ref[...],
        dimension_numbers=(((1,), (1,)), ((), ())),
        preferred_element_type=jnp.float32,
    )
    out_ref[...] = acc + b_ref[...]


def _build_table(emb, W, b):
    grid = VOCAB // _ROW_BLOCK
    return pl.pallas_call(
        _table_body,
        grid=(grid,),
        in_specs=[
            pl.BlockSpec((_ROW_BLOCK, DIM), lambda i: (i, 0)),
            pl.BlockSpec((N_OUT, DIM), lambda i: (0, 0)),
            pl.BlockSpec((1, N_OUT), lambda i: (0, 0)),
        ],
        out_specs=pl.BlockSpec((_ROW_BLOCK, N_OUT), lambda i: (i, 0)),
        out_shape=jax.ShapeDtypeStruct((VOCAB, N_OUT), jnp.float32),
    )(emb, W, b.reshape(1, N_OUT))


@functools.cache
def _make_gather(n_idx):
    NC, NS = 2, 16
    NW = NC * NS
    b_per_w = n_idx // NW
    chunk = 128
    nbuf = 2
    n_chunks = b_per_w // chunk
    mesh = plsc.VectorSubcoreMesh(core_axis_name="c", subcore_axis_name="s")

    stage = VOCAB // NS // 8 * 8

    @functools.partial(
        pl.kernel,
        mesh=mesh,
        out_type=jax.ShapeDtypeStruct((n_idx, N_OUT), jnp.float32),
        scratch_types=[
            pltpu.VMEM_SHARED((VOCAB, N_OUT), jnp.float32),
            pltpu.VMEM((b_per_w,), jnp.int32),
            *[pltpu.VMEM((chunk, N_OUT), jnp.float32) for _ in range(nbuf)],
            *[pltpu.SemaphoreType.DMA for _ in range(2 * nbuf)],
        ],
    )
    def gather_k(table_hbm, idx_hbm, out_hbm, table_spm, idx_v, *bufs_and_sems):
        rows = bufs_and_sems[:nbuf]
        gsem = bufs_and_sems[nbuf:2 * nbuf]
        wsem = bufs_and_sems[2 * nbuf:]
        sid = lax.axis_index("s")
        wid = sid * NC + lax.axis_index("c")
        base = pl.multiple_of(wid * b_per_w, 8)

        soff = pl.multiple_of(sid * stage, 8)
        pltpu.sync_copy(
            table_hbm.at[pl.ds(soff, stage)], table_spm.at[pl.ds(soff, stage)]
        )

        @pl.when(sid == NS - 1)
        def _():
            rem = pl.multiple_of(NS * stage, 8)
            pltpu.sync_copy(
                table_hbm.at[pl.ds(rem, VOCAB - NS * stage)],
                table_spm.at[pl.ds(rem, VOCAB - NS * stage)],
            )

        pltpu.sync_copy(idx_hbm.at[pl.ds(base, b_per_w)], idx_v)
        plsc.subcore_barrier()

        def gather_chunk(c, b):
            off = pl.multiple_of(c * chunk, 8)
            return pltpu.make_async_copy(
                table_spm.at[idx_v.at[pl.ds(off, chunk)]], rows[b], gsem[b]
            )

        def write_chunk(c, b):
            off = pl.multiple_of(base + c * chunk, 8)
            return pltpu.make_async_copy(
                rows[b], out_hbm.at[pl.ds(off, chunk)], wsem[b]
            )

        for b in range(nbuf):
            gather_chunk(b, b).start()
        for c in range(n_chunks):
            b = c % nbuf
            gather_chunk(c, b).wait()
            write_chunk(c, b).start()
            if c + nbuf < n_chunks:
                write_chunk(c, b).wait()
                gather_chunk(c + nbuf, b).start()
        for c in range(max(0, n_chunks - nbuf), n_chunks):
            write_chunk(c, c % nbuf).wait()

    return gather_k


def kernel(x, emb, W, b):
    n_batch, seq = x.shape
    table = _build_table(emb, W, b)
    idx = x.T.reshape(-1).astype(jnp.int32)
    out = _make_gather(idx.shape[0])(table, idx)
    return out.reshape(seq, n_batch, N_OUT).transpose(1, 0, 2)


# speedup vs baseline: 4.0620x; 1.0028x over previous
import functools

import jax
import jax.numpy as jnp
from jax import lax
from jax.experimental import pallas as pl
from jax.experimental.pallas import tpu as pltpu
from jax.experimental.pallas import tpu_sc as plsc

VOCAB = 10000
DIM = 128
N_OUT = 128

_ROW_BLOCK = 2000


def _table_body(emb_ref, w_ref, b_ref, out_ref):
    acc = lax.dot_general(
        emb_ref[...], w_ref[...],
        dimension_numbers=(((1,), (1,)), ((), ())),
        preferred_element_type=jnp.float32,
    )
    out_ref[...] = acc + b_ref[...]


def _build_table(emb, W, b):
    grid = VOCAB // _ROW_BLOCK
    return pl.pallas_call(
        _table_body,
        grid=(grid,),
        in_specs=[
            pl.BlockSpec((_ROW_BLOCK, DIM), lambda i: (i, 0)),
            pl.BlockSpec((N_OUT, DIM), lambda i: (0, 0)),
            pl.BlockSpec((1, N_OUT), lambda i: (0, 0)),
        ],
        out_specs=pl.BlockSpec((_ROW_BLOCK, N_OUT), lambda i: (i, 0)),
        out_shape=jax.ShapeDtypeStruct((VOCAB, N_OUT), jnp.float32),
    )(emb, W, b.reshape(1, N_OUT))


@functools.cache
def _make_gather(n_idx):
    NC, NS = 2, 16
    NW = NC * NS
    b_per_w = n_idx // NW
    chunk = 160
    nbuf = 2
    n_chunks = b_per_w // chunk
    mesh = plsc.VectorSubcoreMesh(core_axis_name="c", subcore_axis_name="s")

    stage = VOCAB // NS // 8 * 8

    @functools.partial(
        pl.kernel,
        mesh=mesh,
        out_type=jax.ShapeDtypeStruct((n_idx, N_OUT), jnp.float32),
        scratch_types=[
            pltpu.VMEM_SHARED((VOCAB, N_OUT), jnp.float32),
            pltpu.VMEM((b_per_w,), jnp.int32),
            *[pltpu.VMEM((chunk, N_OUT), jnp.float32) for _ in range(nbuf)],
            *[pltpu.SemaphoreType.DMA for _ in range(2 * nbuf)],
        ],
    )
    def gather_k(table_hbm, idx_hbm, out_hbm, table_spm, idx_v, *bufs_and_sems):
        rows = bufs_and_sems[:nbuf]
        gsem = bufs_and_sems[nbuf:2 * nbuf]
        wsem = bufs_and_sems[2 * nbuf:]
        sid = lax.axis_index("s")
        wid = sid * NC + lax.axis_index("c")
        base = pl.multiple_of(wid * b_per_w, 8)

        soff = pl.multiple_of(sid * stage, 8)
        pltpu.sync_copy(
            table_hbm.at[pl.ds(soff, stage)], table_spm.at[pl.ds(soff, stage)]
        )

        @pl.when(sid == NS - 1)
        def _():
            rem = pl.multiple_of(NS * stage, 8)
            pltpu.sync_copy(
                table_hbm.at[pl.ds(rem, VOCAB - NS * stage)],
                table_spm.at[pl.ds(rem, VOCAB - NS * stage)],
            )

        pltpu.sync_copy(idx_hbm.at[pl.ds(base, b_per_w)], idx_v)
        plsc.subcore_barrier()

        def gather_chunk(c, b):
            off = pl.multiple_of(c * chunk, 8)
            return pltpu.make_async_copy(
                table_spm.at[idx_v.at[pl.ds(off, chunk)]], rows[b], gsem[b]
            )

        def write_chunk(c, b):
            off = pl.multiple_of(base + c * chunk, 8)
            return pltpu.make_async_copy(
                rows[b], out_hbm.at[pl.ds(off, chunk)], wsem[b]
            )

        for b in range(nbuf):
            gather_chunk(b, b).start()
        for c in range(n_chunks):
            b = c % nbuf
            gather_chunk(c, b).wait()
            write_chunk(c, b).start()
            if c + nbuf < n_chunks:
                write_chunk(c, b).wait()
                gather_chunk(c + nbuf, b).start()
        for c in range(max(0, n_chunks - nbuf), n_chunks):
            write_chunk(c, c % nbuf).wait()

    return gather_k


def kernel(x, emb, W, b):
    n_batch, seq = x.shape
    table = _build_table(emb, W, b)
    idx = x.T.reshape(-1).astype(jnp.int32)
    out = _make_gather(idx.shape[0])(table, idx)
    return out.reshape(seq, n_batch, N_OUT).transpose(1, 0, 2)


# speedup vs baseline: 4.1045x; 1.0105x over previous
import functools

import jax
import jax.numpy as jnp
from jax import lax
from jax.experimental import pallas as pl
from jax.experimental.pallas import tpu as pltpu
from jax.experimental.pallas import tpu_sc as plsc

VOCAB = 10000
DIM = 128
N_OUT = 128

_ROW_BLOCK = 2000


def _table_body(emb_ref, w_ref, b_ref, out_ref):
    acc = lax.dot_general(
        emb_ref[...], w_ref[...],
        dimension_numbers=(((1,), (1,)), ((), ())),
        preferred_element_type=jnp.float32,
    )
    out_ref[...] = acc + b_ref[...]


def _build_table(emb, W, b):
    grid = VOCAB // _ROW_BLOCK
    return pl.pallas_call(
        _table_body,
        grid=(grid,),
        in_specs=[
            pl.BlockSpec((_ROW_BLOCK, DIM), lambda i: (i, 0)),
            pl.BlockSpec((N_OUT, DIM), lambda i: (0, 0)),
            pl.BlockSpec((1, N_OUT), lambda i: (0, 0)),
        ],
        out_specs=pl.BlockSpec((_ROW_BLOCK, N_OUT), lambda i: (i, 0)),
        out_shape=jax.ShapeDtypeStruct((VOCAB, N_OUT), jnp.float32),
    )(emb, W, b.reshape(1, N_OUT))


@functools.cache
def _make_gather(n_idx):
    NC, NS = 2, 16
    NW = NC * NS
    b_per_w = n_idx // NW
    chunk = 160
    nbuf = 2
    n_chunks = b_per_w // chunk
    mesh = plsc.VectorSubcoreMesh(core_axis_name="c", subcore_axis_name="s")

    stage = VOCAB // NS // 8 * 8

    @functools.partial(
        pl.kernel,
        mesh=mesh,
        out_type=jax.ShapeDtypeStruct((n_idx, N_OUT), jnp.float32),
        scratch_types=[
            pltpu.VMEM_SHARED((VOCAB, N_OUT), jnp.float32),
            pltpu.VMEM((b_per_w,), jnp.int32),
            *[pltpu.VMEM((chunk, N_OUT), jnp.float32) for _ in range(nbuf)],
            *[pltpu.SemaphoreType.DMA for _ in range(2 * nbuf)],
        ],
    )
    def gather_k(table_hbm, idx_hbm, out_hbm, table_spm, idx_v, *bufs_and_sems):
        rows = bufs_and_sems[:nbuf]
        gsem = bufs_and_sems[nbuf:2 * nbuf]
        wsem = bufs_and_sems[2 * nbuf:]
        sid = lax.axis_index("s")
        wid = sid * NC + lax.axis_index("c")
        base = pl.multiple_of(wid * b_per_w, 8)

        soff = pl.multiple_of(sid * stage, 8)
        pltpu.sync_copy(
            table_hbm.at[pl.ds(soff, stage)], table_spm.at[pl.ds(soff, stage)]
        )

        @pl.when(sid == NS - 1)
        def _():
            rem = pl.multiple_of(NS * stage, 8)
            pltpu.sync_copy(
                table_hbm.at[pl.ds(rem, VOCAB - NS * stage)],
                table_spm.at[pl.ds(rem, VOCAB - NS * stage)],
            )

        pltpu.sync_copy(idx_hbm.at[pl.ds(base, b_per_w)], idx_v)
        plsc.subcore_barrier()

        def gather_chunk(c, b):
            off = pl.multiple_of(c * chunk, 8)
            return pltpu.make_async_copy(
                table_spm.at[idx_v.at[pl.ds(off, chunk)]], rows[b], gsem[b]
            )

        def write_chunk(c, b):
            off = pl.multiple_of(base + c * chunk, 8)
            return pltpu.make_async_copy(
                rows[b], out_hbm.at[pl.ds(off, chunk)], wsem[b]
            )

        for b in range(nbuf):
            gather_chunk(b, b).start()

        def step(g, carry):
            for b in range(nbuf):
                c = g * nbuf + b
                gather_chunk(c, b).wait()
                write_chunk(c, b).start()

                @pl.when(c + nbuf < n_chunks)
                def _():
                    write_chunk(c, b).wait()
                    gather_chunk(c + nbuf, b).start()

            return carry

        lax.fori_loop(0, n_chunks // nbuf, step, 0)
        for c in range(n_chunks - nbuf, n_chunks):
            write_chunk(c, c % nbuf).wait()

    return gather_k


def kernel(x, emb, W, b):
    n_batch, seq = x.shape
    table = _build_table(emb, W, b)
    idx = x.T.reshape(-1).astype(jnp.int32)
    out = _make_gather(idx.shape[0])(table, idx)
    return out.reshape(seq, n_batch, N_OUT).transpose(1, 0, 2)


# speedup vs baseline: 4.2353x; 1.0319x over previous
import functools

import jax
import jax.numpy as jnp
from jax import lax
from jax.experimental import pallas as pl
from jax.experimental.pallas import tpu as pltpu
from jax.experimental.pallas import tpu_sc as plsc

VOCAB = 10000
DIM = 128
N_OUT = 128

_ROW_BLOCK = 5000


def _table_body(emb_ref, w_ref, b_ref, out_ref):
    acc = lax.dot_general(
        emb_ref[...], w_ref[...],
        dimension_numbers=(((1,), (1,)), ((), ())),
        preferred_element_type=jnp.float32,
    )
    out_ref[...] = acc + b_ref[...]


def _build_table(emb, W, b):
    grid = VOCAB // _ROW_BLOCK
    return pl.pallas_call(
        _table_body,
        grid=(grid,),
        in_specs=[
            pl.BlockSpec((_ROW_BLOCK, DIM), lambda i: (i, 0)),
            pl.BlockSpec((N_OUT, DIM), lambda i: (0, 0)),
            pl.BlockSpec((1, N_OUT), lambda i: (0, 0)),
        ],
        out_specs=pl.BlockSpec((_ROW_BLOCK, N_OUT), lambda i: (i, 0)),
        out_shape=jax.ShapeDtypeStruct((VOCAB, N_OUT), jnp.float32),
    )(emb, W, b.reshape(1, N_OUT))


@functools.cache
def _make_gather(n_idx):
    NC, NS = 2, 16
    NW = NC * NS
    b_per_w = n_idx // NW
    chunk = 160
    nbuf = 2
    n_chunks = b_per_w // chunk
    mesh = plsc.VectorSubcoreMesh(core_axis_name="c", subcore_axis_name="s")

    stage = VOCAB // NS // 8 * 8

    @functools.partial(
        pl.kernel,
        mesh=mesh,
        out_type=jax.ShapeDtypeStruct((n_idx, N_OUT), jnp.float32),
        scratch_types=[
            pltpu.VMEM_SHARED((VOCAB, N_OUT), jnp.float32),
            pltpu.VMEM((b_per_w,), jnp.int32),
            *[pltpu.VMEM((chunk, N_OUT), jnp.float32) for _ in range(nbuf)],
            *[pltpu.SemaphoreType.DMA for _ in range(2 * nbuf)],
        ],
    )
    def gather_k(table_hbm, idx_hbm, out_hbm, table_spm, idx_v, *bufs_and_sems):
        rows = bufs_and_sems[:nbuf]
        gsem = bufs_and_sems[nbuf:2 * nbuf]
        wsem = bufs_and_sems[2 * nbuf:]
        sid = lax.axis_index("s")
        wid = sid * NC + lax.axis_index("c")
        base = pl.multiple_of(wid * b_per_w, 8)

        soff = pl.multiple_of(sid * stage, 8)
        pltpu.sync_copy(
            table_hbm.at[pl.ds(soff, stage)], table_spm.at[pl.ds(soff, stage)]
        )

        @pl.when(sid == NS - 1)
        def _():
            rem = pl.multiple_of(NS * stage, 8)
            pltpu.sync_copy(
                table_hbm.at[pl.ds(rem, VOCAB - NS * stage)],
                table_spm.at[pl.ds(rem, VOCAB - NS * stage)],
            )

        pltpu.sync_copy(idx_hbm.at[pl.ds(base, b_per_w)], idx_v)
        plsc.subcore_barrier()

        def gather_chunk(c, b):
            off = pl.multiple_of(c * chunk, 8)
            return pltpu.make_async_copy(
                table_spm.at[idx_v.at[pl.ds(off, chunk)]], rows[b], gsem[b]
            )

        def write_chunk(c, b):
            off = pl.multiple_of(base + c * chunk, 8)
            return pltpu.make_async_copy(
                rows[b], out_hbm.at[pl.ds(off, chunk)], wsem[b]
            )

        for b in range(nbuf):
            gather_chunk(b, b).start()

        def step(g, carry):
            for b in range(nbuf):
                c = g * nbuf + b
                gather_chunk(c, b).wait()
                write_chunk(c, b).start()

                @pl.when(c + nbuf < n_chunks)
                def _():
                    write_chunk(c, b).wait()
                    gather_chunk(c + nbuf, b).start()

            return carry

        lax.fori_loop(0, n_chunks // nbuf, step, 0)
        for c in range(n_chunks - nbuf, n_chunks):
            write_chunk(c, c % nbuf).wait()

    return gather_k


def kernel(x, emb, W, b):
    n_batch, seq = x.shape
    table = _build_table(emb, W, b)
    idx = x.T.reshape(-1).astype(jnp.int32)
    out = _make_gather(idx.shape[0])(table, idx)
    return out.reshape(seq, n_batch, N_OUT).transpose(1, 0, 2)


# speedup vs baseline: 4.3714x; 1.0321x over previous
import functools

import jax
import jax.numpy as jnp
from jax import lax
from jax.experimental import pallas as pl
from jax.experimental.pallas import tpu as pltpu
from jax.experimental.pallas import tpu_sc as plsc

VOCAB = 10000
DIM = 128
N_OUT = 128

_ROW_BLOCK = 5000


def _table_body(emb_ref, w_ref, b_ref, out_ref):
    acc = lax.dot_general(
        emb_ref[...], w_ref[...],
        dimension_numbers=(((1,), (1,)), ((), ())),
        preferred_element_type=jnp.float32,
    )
    out_ref[...] = acc + b_ref[...]


def _build_table(emb, W, b):
    grid = VOCAB // _ROW_BLOCK
    return pl.pallas_call(
        _table_body,
        grid=(grid,),
        in_specs=[
            pl.BlockSpec((_ROW_BLOCK, DIM), lambda i: (i, 0)),
            pl.BlockSpec((N_OUT, DIM), lambda i: (0, 0)),
            pl.BlockSpec((1, N_OUT), lambda i: (0, 0)),
        ],
        out_specs=pl.BlockSpec((_ROW_BLOCK, N_OUT), lambda i: (i, 0)),
        out_shape=jax.ShapeDtypeStruct((VOCAB, N_OUT), jnp.float32),
    )(emb, W, b.reshape(1, N_OUT))


@functools.cache
def _make_gather(n_idx):
    NC, NS = 2, 16
    NW = NC * NS
    b_per_w = n_idx // NW
    chunk = 160
    nbuf = 2
    n_chunks = b_per_w // chunk
    mesh = plsc.VectorSubcoreMesh(core_axis_name="c", subcore_axis_name="s")

    stage = VOCAB // NS // 8 * 8

    @functools.partial(
        pl.kernel,
        mesh=mesh,
        out_type=jax.ShapeDtypeStruct((n_idx, N_OUT), jnp.float32),
        scratch_types=[
            pltpu.VMEM_SHARED((VOCAB, N_OUT), jnp.float32),
            pltpu.VMEM((b_per_w,), jnp.int32),
            *[pltpu.VMEM((chunk, N_OUT), jnp.float32) for _ in range(nbuf)],
            *[pltpu.SemaphoreType.DMA for _ in range(2 * nbuf + 1)],
        ],
    )
    def gather_k(table_hbm, idx_hbm, out_hbm, table_spm, idx_v, *bufs_and_sems):
        rows = bufs_and_sems[:nbuf]
        gsem = bufs_and_sems[nbuf:2 * nbuf]
        wsem = bufs_and_sems[2 * nbuf:3 * nbuf]
        ssem = bufs_and_sems[3 * nbuf]
        sid = lax.axis_index("s")
        wid = sid * NC + lax.axis_index("c")
        base = pl.multiple_of(wid * b_per_w, 8)

        soff = pl.multiple_of(sid * stage, 8)
        rem = pl.multiple_of(NS * stage, 8)
        stage_cp = pltpu.make_async_copy(
            table_hbm.at[pl.ds(soff, stage)],
            table_spm.at[pl.ds(soff, stage)],
            ssem,
        )
        rem_cp = pltpu.make_async_copy(
            table_hbm.at[pl.ds(rem, VOCAB - NS * stage)],
            table_spm.at[pl.ds(rem, VOCAB - NS * stage)],
            ssem,
        )
        stage_cp.start()

        @pl.when(sid == NS - 1)
        def _():
            rem_cp.start()

        pltpu.sync_copy(idx_hbm.at[pl.ds(base, b_per_w)], idx_v)

        def gather_chunk(c, b, src=None):
            off = pl.multiple_of(c * chunk, 8)
            src = table_spm if src is None else src
            return pltpu.make_async_copy(
                src.at[idx_v.at[pl.ds(off, chunk)]], rows[b], gsem[b]
            )

        def write_chunk(c, b):
            off = pl.multiple_of(base + c * chunk, 8)
            return pltpu.make_async_copy(
                rows[b], out_hbm.at[pl.ds(off, chunk)], wsem[b]
            )

        for b in range(nbuf):
            gather_chunk(b, b, src=table_hbm).start()

        stage_cp.wait()

        @pl.when(sid == NS - 1)
        def _():
            rem_cp.wait()

        plsc.subcore_barrier()

        def step(g, carry):
            for b in range(nbuf):
                c = g * nbuf + b
                gather_chunk(c, b).wait()
                write_chunk(c, b).start()

                @pl.when(c + nbuf < n_chunks)
                def _():
                    write_chunk(c, b).wait()
                    gather_chunk(c + nbuf, b).start()

            return carry

        lax.fori_loop(0, n_chunks // nbuf, step, 0)
        for c in range(n_chunks - nbuf, n_chunks):
            write_chunk(c, c % nbuf).wait()

    return gather_k


def kernel(x, emb, W, b):
    n_batch, seq = x.shape
    table = _build_table(emb, W, b)
    idx = x.T.reshape(-1).astype(jnp.int32)
    out = _make_gather(idx.shape[0])(table, idx)
    return out.reshape(seq, n_batch, N_OUT).transpose(1, 0, 2)
